# baseline (speedup 1.0000x reference)
import functools

import jax
import jax.numpy as jnp
from jax import lax
from jax.experimental import pallas as pl
from jax.experimental.pallas import tpu as pltpu
from jax.experimental.pallas import tpu_sc as plsc

N_NODES = 10000
D = 256
DH = 128

NC = 2
NS = 16
K = 128

NCHUNK = 81
EPT = NCHUNK * K
E_PAD = EPT * NS
ROWS_PT = N_NODES // NS
RBITS = 14


def _sc_aggregate(xs, packed_r, ew_r):
  mesh = plsc.VectorSubcoreMesh(core_axis_name="c", subcore_axis_name="s")

  @functools.partial(
      pl.kernel,
      out_type=jax.ShapeDtypeStruct((N_NODES, D), jnp.float32),
      mesh=mesh,
      scratch_types=[
          pltpu.VMEM((3, K), jnp.int32),
          pltpu.VMEM((3, K), jnp.float32),
          pltpu.VMEM((3, K), jnp.int32),
          pltpu.VMEM((K, DH), jnp.float32),
          pltpu.VMEM((K, DH), jnp.float32),
          pltpu.VMEM((K, DH), jnp.float32),
          pltpu.VMEM_SHARED((N_NODES, DH), jnp.float32),
          pltpu.SemaphoreType.DMA,
          pltpu.SemaphoreType.DMA,
          pltpu.SemaphoreType.DMA,
          pltpu.SemaphoreType.DMA,
          pltpu.SemaphoreType.DMA,
          pltpu.SemaphoreType.DMA,
          pltpu.SemaphoreType.DMA,
          pltpu.SemaphoreType.DMA,
          pltpu.SemaphoreType.DMA,
      ],
  )
  def agg_kernel(xs_hbm, packed_hbm, ew_hbm, out_hbm,
                 ed, wvd, rowk, buf0, buf1, buf2, acc,
                 gs0, gs1, gs2, ss0, ss1, ss2, es0, es1, es2):
    bufs = (buf0, buf1, buf2)
    gsem = (gs0, gs1, gs2)
    ssem = (ss0, ss1, ss2)
    esem = (es0, es1, es2)
    c = lax.axis_index("c")
    s = lax.axis_index("s")

    mask = jnp.full((16,), (1 << RBITS) - 1, jnp.int32)

    def fire_edata(k, r):
      pltpu.async_copy(packed_hbm.at[s].at[k], ed.at[r], esem[r])
      pltpu.async_copy(ew_hbm.at[s].at[k], wvd.at[r], esem[r])

    def wait_edata(r):
      pltpu.make_async_copy(packed_hbm.at[s].at[0], ed.at[r], esem[r]).wait()
      pltpu.make_async_copy(ew_hbm.at[s].at[0], wvd.at[r], esem[r]).wait()

    def unpack(r):
      for g in range(K // 16):
        v = ed[r, pl.ds(g * 16, 16)]
        rowk[r, pl.ds(g * 16, 16)] = lax.shift_right_logical(v, RBITS)
        ed[r, pl.ds(g * 16, 16)] = v & mask

    def fire_gather(k, r):
      pltpu.async_copy(xs_hbm.at[c].at[ed.at[r]], bufs[r], gsem[r])

    def wait_gather(r):
      pltpu.make_async_copy(xs_hbm.at[c].at[pl.ds(0, K)], bufs[r],
                            gsem[r]).wait()

    def fire_scatter(k, r):
      pltpu.async_copy(bufs[r], acc.at[rowk.at[r]], ssem[r], add=True)

    def wait_scatter(r):
      pltpu.make_async_copy(bufs[r], acc.at[pl.ds(0, K)], ssem[r]).wait()

    def scale(k, r):
      b = bufs[r]
      for g in range(K // 16):
        w16 = wvd[r, pl.ds(g * 16, 16)]
        for e in range(16):
          w = w16[e]
          rr = g * 16 + e
          for j in range(DH // 16):
            b[rr, pl.ds(j * 16, 16)] = b[rr, pl.ds(j * 16, 16)] * w

    def zrow(i, _):
      for j in range(DH // 16):
        buf0[i, pl.ds(j * 16, 16)] = jnp.zeros((16,), jnp.float32)
      return 0
    lax.fori_loop(0, K, zrow, 0)
    base = s * ROWS_PT
    nz = ROWS_PT // K
    for kk in range(nz):
      pltpu.sync_copy(buf0, acc.at[pl.ds(base + kk * K, K)])
    pltpu.sync_copy(buf0.at[pl.ds(0, ROWS_PT - nz * K)],
                    acc.at[pl.ds(base + nz * K, ROWS_PT - nz * K)])
    plsc.subcore_barrier()

    fire_edata(0, 0)
    fire_edata(1, 1)
    wait_edata(0)
    unpack(0)
    fire_gather(0, 0)

    def triple_body(t, _):
      for j in range(3):
        k = 3 * t + j
        r = j
        r1 = (j + 1) % 3
        r2 = (j + 2) % 3
        @pl.when(k + 1 < NCHUNK)
        def _():
          @pl.when(k >= 2)
          def _():
            wait_scatter(r1)
          wait_edata(r1)
          unpack(r1)
          fire_gather(k + 1, r1)
        wait_gather(r)
        scale(k, r)
        fire_scatter(k, r)
        @pl.when(k + 2 < NCHUNK)
        def _():
          fire_edata(k + 2, r2)
      return 0
    lax.fori_loop(0, NCHUNK // 3, triple_body, 0)
    wait_scatter(0)
    wait_scatter(1)
    wait_scatter(2)
    plsc.subcore_barrier()

    nfull = N_NODES // K
    def drain_chunk(t, _):
      cid = s + NS * t
      @pl.when(cid < nfull)
      def _():
        r0 = cid * K
        pltpu.sync_copy(acc.at[pl.ds(r0, K)], buf0)
        pltpu.sync_copy(buf0, out_hbm.at[pl.ds(r0, K), pl.ds(c * DH, DH)])
      return 0
    lax.fori_loop(0, (nfull + NS - 1) // NS, drain_chunk, 0)
    tail = N_NODES - nfull * K
    @pl.when(s == NS - 1)
    def _():
      pltpu.sync_copy(acc.at[pl.ds(nfull * K, tail)], buf0.at[pl.ds(0, tail)])
      pltpu.sync_copy(buf0.at[pl.ds(0, tail)],
                      out_hbm.at[pl.ds(nfull * K, tail), pl.ds(c * DH, DH)])

  return agg_kernel(xs, packed_r, ew_r)


def _tc_matmul_bias_relu(agg, W, b2):
  BM = 1000

  def mm_body(a_ref, w_ref, b_ref, o_ref):
    h = lax.dot_general(a_ref[...], w_ref[...],
                        (((1,), (1,)), ((), ())),
                        preferred_element_type=jnp.float32)
    o_ref[...] = jnp.maximum(h + b_ref[...], 0.0)

  return pl.pallas_call(
      mm_body,
      out_shape=jax.ShapeDtypeStruct((N_NODES, D), jnp.float32),
      grid=(N_NODES // BM,),
      in_specs=[
          pl.BlockSpec((BM, D), lambda i: (i, 0)),
          pl.BlockSpec((D, D), lambda i: (0, 0)),
          pl.BlockSpec((1, D), lambda i: (0, 0)),
      ],
      out_specs=pl.BlockSpec((BM, D), lambda i: (i, 0)),
  )(agg, W, b2)


def kernel(x, edge_index, edge_weight, W, b):
  row = edge_index[0].astype(jnp.int32)
  col = edge_index[1].astype(jnp.int32)
  ew = edge_weight.astype(jnp.float32)

  e = row.shape[0]
  pad = E_PAD - e
  packed = (row << RBITS) | col
  packed_p = jnp.concatenate([packed, jnp.zeros((pad,), jnp.int32)])
  ew_p = jnp.concatenate([ew, jnp.zeros((pad,), jnp.float32)])

  packed_r = packed_p.reshape(NS, NCHUNK, K)
  ew_r = ew_p.reshape(NS, NCHUNK, K)

  xs = jnp.stack([x[:, :DH], x[:, DH:]])

  agg = _sc_aggregate(xs, packed_r, ew_r)
  return _tc_matmul_bias_relu(agg, W, b[None, :])
